# baseline (speedup 1.0000x reference)
import functools

import jax
import jax.numpy as jnp
import numpy as np
from jax import lax
from jax.experimental import pallas as pl
from jax.experimental.pallas import tpu as pltpu
from jax.experimental.pallas import tpu_sc as plsc

_NC = 2
_NS = 16
_NW = _NC * _NS
_H = jax.lax.Precision.HIGHEST


def _tc_matmul(x, Wx):
    n, d = x.shape
    b = 1000

    def body(x_ref, w_ref, o_ref):
        o_ref[...] = jnp.dot(x_ref[...], w_ref[...],
                             precision=_H).astype(jnp.bfloat16)

    return pl.pallas_call(
        body,
        grid=(n // b,),
        in_specs=[
            pl.BlockSpec((b, d), lambda i: (i, 0)),
            pl.BlockSpec(Wx.shape, lambda i: (0, 0)),
        ],
        out_specs=pl.BlockSpec((b, Wx.shape[1]), lambda i: (i, 0)),
        out_shape=jax.ShapeDtypeStruct((n, Wx.shape[1]), jnp.bfloat16),
    )(x, Wx)


def _sc_gather_rows(xw, src):
    e = src.shape[0]
    d = xw.shape[1]
    c = 400
    per_w = e // _NW
    steps = per_w // c
    mesh = plsc.VectorSubcoreMesh(core_axis_name="c", subcore_axis_name="s")

    @functools.partial(
        pl.kernel,
        out_type=jax.ShapeDtypeStruct((e, d), jnp.bfloat16),
        mesh=mesh,
        scratch_types=[
            pltpu.VMEM((c,), jnp.int32),
            pltpu.VMEM((c,), jnp.int32),
            pltpu.VMEM((c, d), jnp.bfloat16),
            pltpu.VMEM((c, d), jnp.bfloat16),
            pltpu.SemaphoreType.DMA,
            pltpu.SemaphoreType.DMA,
        ],
        compiler_params=pltpu.CompilerParams(use_tc_tiling_on_sc=False),
    )
    def k(xw_hbm, src_hbm, xwg_hbm, idx0, idx1, rows0, rows1, sem0, sem1):
        wid = lax.axis_index("s") * _NC + lax.axis_index("c")
        base = wid * per_w
        idxb = (idx0, idx1)
        rowsb = (rows0, rows1)
        semb = (sem0, sem1)

        pltpu.sync_copy(src_hbm.at[pl.ds(base, c)], idx0)
        handles = {0: pltpu.async_copy(xw_hbm.at[idx0], rows0, sem0)}
        for j in range(steps):
            b = j % 2
            handles[j].wait()
            if j + 1 < steps:
                nb = (j + 1) % 2
                pltpu.sync_copy(src_hbm.at[pl.ds(base + (j + 1) * c, c)],
                                idxb[nb])
                handles[j + 1] = pltpu.async_copy(xw_hbm.at[idxb[nb]],
                                                  rowsb[nb], semb[nb])
            pltpu.sync_copy(rowsb[b], xwg_hbm.at[pl.ds(base + j * c, c)])

    return k(xw, src)


def _sc_gather_dvec(pos_flat, src, dst):
    e = src.shape[0]
    npts8 = pos_flat.shape[0]
    c = 2000
    per_w = e // _NW
    steps = per_w // c
    mesh = plsc.VectorSubcoreMesh(core_axis_name="c", subcore_axis_name="s")

    @functools.partial(
        pl.kernel,
        out_type=jax.ShapeDtypeStruct((3, _NW, per_w), jnp.float32),
        mesh=mesh,
        scratch_types=[
            pltpu.VMEM((npts8,), jnp.float32),
            pltpu.VMEM((c,), jnp.int32),
            pltpu.VMEM((c,), jnp.int32),
            pltpu.VMEM((c,), jnp.float32),
            pltpu.VMEM((c,), jnp.float32),
            pltpu.VMEM((c,), jnp.float32),
        ],
        compiler_params=pltpu.CompilerParams(use_tc_tiling_on_sc=False,
                                             needs_layout_passes=False),
    )
    def k(pos_hbm, src_hbm, dst_hbm, dv_hbm, pos_v, idxs_v, idxd_v,
          dx_v, dy_v, dz_v):
        wid = lax.axis_index("s") * _NC + lax.axis_index("c")
        base = wid * per_w
        pltpu.sync_copy(pos_hbm, pos_v)

        @pl.loop(0, steps)
        def _(i):
            off = base + i * c
            pltpu.sync_copy(src_hbm.at[pl.ds(off, c)], idxs_v)
            pltpu.sync_copy(dst_hbm.at[pl.ds(off, c)], idxd_v)

            @pl.loop(0, c // 16)
            def _(g):
                sl = pl.ds(g * 16, 16)
                s8 = idxs_v[sl] * 8
                d8 = idxd_v[sl] * 8
                dx_v.at[sl][...] = (plsc.load_gather(pos_v, [d8])
                                    - plsc.load_gather(pos_v, [s8]))
                dy_v.at[sl][...] = (plsc.load_gather(pos_v, [d8 + 1])
                                    - plsc.load_gather(pos_v, [s8 + 1]))
                dz_v.at[sl][...] = (plsc.load_gather(pos_v, [d8 + 2])
                                    - plsc.load_gather(pos_v, [s8 + 2]))

            pltpu.sync_copy(dx_v, dv_hbm.at[0, wid, pl.ds(i * c, c)])
            pltpu.sync_copy(dy_v, dv_hbm.at[1, wid, pl.ds(i * c, c)])
            pltpu.sync_copy(dz_v, dv_hbm.at[2, wid, pl.ds(i * c, c)])

    return k(pos_flat, src, dst)


def _tc_message(xwg, dxa, dya, dza, W1T, b1T, W2, b2, Wy16):
    e, d = xwg.shape
    nb, _, b = dxa.shape
    s3 = np.float32(np.sqrt(3.0))
    dn = (((0,), (0,)), ((), ()))

    def body(xwg_ref, dx_ref, dy_ref, dz_ref, w1_ref, b1_ref, w2_ref,
             b2_ref, wy_ref, o_ref):
        dx = dx_ref[0]
        dy = dy_ref[0]
        dz = dz_ref[0]
        d2 = dx * dx + dy * dy + dz * dz
        ln = jnp.maximum(jnp.sqrt(d2), 1e-8)
        inv = 1.0 / ln
        ex = dx * inv
        ey = dy * inv
        ez = dz * inv
        Yl = jnp.concatenate(
            [
                jnp.ones_like(ex),
                ex, ey, ez,
                s3 * ex * ey,
                s3 * ey * ez,
                0.5 * (3.0 * ez * ez - 1.0),
                s3 * ex * ez,
                (s3 / 2.0) * (ex * ex - ey * ey),
                jnp.zeros((7, b), jnp.float32),
            ],
            axis=0,
        )
        yw = lax.dot_general(Yl, wy_ref[...], dn, precision=None)
        hl = jax.nn.silu(w1_ref[...] * ln + b1_ref[...])
        w = lax.dot_general(hl, w2_ref[...], dn, precision=None) + b2_ref[...]
        xg = xwg_ref[...].astype(jnp.float32)
        o_ref[...] = xg * (yw * w)

    return pl.pallas_call(
        body,
        grid=(nb,),
        in_specs=[
            pl.BlockSpec((b, d), lambda i: (i, 0)),
            pl.BlockSpec((1, 1, b), lambda i: (i, 0, 0)),
            pl.BlockSpec((1, 1, b), lambda i: (i, 0, 0)),
            pl.BlockSpec((1, 1, b), lambda i: (i, 0, 0)),
            pl.BlockSpec((64, 1), lambda i: (0, 0)),
            pl.BlockSpec((64, 1), lambda i: (0, 0)),
            pl.BlockSpec((64, 128), lambda i: (0, 0)),
            pl.BlockSpec((1, 128), lambda i: (0, 0)),
            pl.BlockSpec((16, 128), lambda i: (0, 0)),
        ],
        out_specs=pl.BlockSpec((b, d), lambda i: (i, 0)),
        out_shape=jax.ShapeDtypeStruct((e, d), jnp.float32),
    )(xwg, dxa, dya, dza, W1T, b1T, W2, b2, Wy16)


def _sc_scatter(m_list, dst, n, e0):
    nm = len(m_list)
    ec, d = m_list[0].shape
    c = 80
    per_sub = ec // _NC // _NS
    steps = per_sub // c
    wb_rows = 1000
    zb = 40
    mesh = plsc.VectorSubcoreMesh(core_axis_name="c", subcore_axis_name="s")

    @functools.partial(
        pl.kernel,
        out_type=jax.ShapeDtypeStruct((_NC, n, d), jnp.float32),
        mesh=mesh,
        scratch_types=[
            pltpu.VMEM((c,), jnp.int32),
            pltpu.VMEM((c,), jnp.int32),
            pltpu.VMEM((c, d), jnp.float32),
            pltpu.VMEM((c, d), jnp.float32),
            pltpu.VMEM((zb, d), jnp.float32),
            pltpu.VMEM_SHARED((n, d), jnp.float32),
            pltpu.SemaphoreType.DMA,
            pltpu.SemaphoreType.DMA,
        ],
    )
    def k(*refs):
        m_hbms = refs[:nm]
        (dst_hbm, out_hbm, idx0, idx1, rows0, rows1, zero_v, acc_sh,
         sem0, sem1) = refs[nm:]
        cid = lax.axis_index("c")
        sid = lax.axis_index("s")
        zvec = jnp.zeros((16,), jnp.float32)
        idxb = (idx0, idx1)
        rowsb = (rows0, rows1)
        semb = (sem0, sem1)

        @pl.loop(0, zb)
        def _(r):
            @pl.loop(0, d // 16)
            def _(j):
                zero_v.at[r, pl.ds(j * 16, 16)][...] = zvec

        @pl.when(sid < n // wb_rows)
        def _():
            @pl.loop(0, wb_rows // zb)
            def _(bk):
                pltpu.sync_copy(zero_v,
                                acc_sh.at[pl.ds(sid * wb_rows + bk * zb, zb)])

        plsc.subcore_barrier()

        local0 = cid * (ec // _NC) + sid * per_sub
        iters = [(m_hbms[mi], mi, i) for mi in range(nm)
                 for i in range(steps)]

        def start_load(j, b):
            m_hbm, mi, i = iters[j]
            loc = local0 + i * c
            gof = e0 + mi * ec + loc
            return (pltpu.async_copy(dst_hbm.at[pl.ds(gof, c)], idxb[b],
                                     semb[b]),
                    pltpu.async_copy(m_hbm.at[pl.ds(loc, c)], rowsb[b],
                                     semb[b]))

        handles = {0: start_load(0, 0)}
        for j in range(len(iters)):
            b = j % 2
            h1, h2 = handles[j]
            h1.wait()
            h2.wait()
            if j + 1 < len(iters):
                handles[j + 1] = start_load(j + 1, (j + 1) % 2)
            pltpu.sync_copy(rowsb[b], acc_sh.at[idxb[b]], add=True)

        plsc.subcore_barrier()

        @pl.when(sid < n // wb_rows)
        def _():
            pltpu.sync_copy(acc_sh.at[pl.ds(sid * wb_rows, wb_rows)],
                            out_hbm.at[cid, pl.ds(sid * wb_rows, wb_rows)])

    return k(*m_list, dst)


def _tc_head(parts_list, Ws, Wns, Wg):
    np_ = len(parts_list)
    _, n, d = parts_list[0].shape
    b = 1000

    def body(*refs):
        p_refs = refs[:np_]
        ws_ref, wns_ref, wg_ref, o_ref = refs[np_:]
        out = p_refs[0][0] + p_refs[0][1]
        for p in p_refs[1:]:
            out = out + p[0] + p[1]
        s = jax.nn.silu(jnp.dot(out, ws_ref[...]))
        ns = jnp.dot(out, wns_ref[...])
        g = jax.nn.sigmoid(jnp.dot(out, wg_ref[...]))
        i0 = lax.broadcasted_iota(jnp.int32, (32, 96), 0)
        i1 = lax.broadcasted_iota(jnp.int32, (32, 96), 1)
        rep = (i0 == i1 // 3).astype(jnp.float32)
        gr = jnp.dot(g, rep, precision=_H)
        o_ref[...] = jnp.concatenate([s, gr * ns], axis=1)

    return pl.pallas_call(
        body,
        grid=(n // b,),
        in_specs=(
            [pl.BlockSpec((2, b, d), lambda i: (0, i, 0))] * np_
            + [
                pl.BlockSpec((128, 32), lambda i: (0, 0)),
                pl.BlockSpec((128, 96), lambda i: (0, 0)),
                pl.BlockSpec((128, 32), lambda i: (0, 0)),
            ]
        ),
        out_specs=pl.BlockSpec((b, d), lambda i: (i, 0)),
        out_shape=jax.ShapeDtypeStruct((n, d), jnp.float32),
    )(*parts_list, Ws, Wns, Wg)


def kernel(x, edge_index, pos, W1, b1, W2, b2, Wx, Wy, Ws, Wns, Wg):
    n = x.shape[0]
    e = edge_index.shape[1]
    be = 2560
    nk = 5
    ec = e // nk
    src = edge_index[0]
    dst = edge_index[1]
    pos_flat = jnp.pad(pos, ((0, 0), (0, 5))).reshape(-1)
    Wy16 = jnp.pad(Wy, ((0, 7), (0, 0)))
    W1T = W1.reshape(-1, 1)
    b1T = b1.reshape(-1, 1)
    b2r = b2.reshape(1, -1)

    xw = _tc_matmul(x, Wx)
    dv = _sc_gather_dvec(pos_flat, src, dst).reshape(3, e)
    ms = []
    for k in range(nk):
        srck = lax.slice(src, (k * ec,), ((k + 1) * ec,))
        xwg = _sc_gather_rows(xw, srck)
        dxa = lax.slice(dv[0], (k * ec,), ((k + 1) * ec,)).reshape(
            ec // be, 1, be)
        dya = lax.slice(dv[1], (k * ec,), ((k + 1) * ec,)).reshape(
            ec // be, 1, be)
        dza = lax.slice(dv[2], (k * ec,), ((k + 1) * ec,)).reshape(
            ec // be, 1, be)
        ms.append(_tc_message(xwg, dxa, dya, dza, W1T, b1T, W2, b2r, Wy16))
    parts_list = [
        _sc_scatter(ms[:4], dst, n, 0),
        _sc_scatter(ms[4:], dst, n, 4 * ec),
    ]
    return _tc_head(parts_list, Ws, Wns, Wg)


# speedup vs baseline: 1.4783x; 1.4783x over previous
import functools

import jax
import jax.numpy as jnp
import numpy as np
from jax import lax
from jax.experimental import pallas as pl
from jax.experimental.pallas import tpu as pltpu
from jax.experimental.pallas import tpu_sc as plsc

_NC = 2
_NS = 16
_NW = _NC * _NS
_H = jax.lax.Precision.HIGHEST


def _tc_matmul(x, Wx):
    n, d = x.shape
    b = 1000

    def body(x_ref, w_ref, o_ref):
        o_ref[...] = jnp.dot(x_ref[...], w_ref[...], precision=_H)

    return pl.pallas_call(
        body,
        grid=(n // b,),
        in_specs=[
            pl.BlockSpec((b, d), lambda i: (i, 0)),
            pl.BlockSpec(Wx.shape, lambda i: (0, 0)),
        ],
        out_specs=pl.BlockSpec((b, Wx.shape[1]), lambda i: (i, 0)),
        out_shape=jax.ShapeDtypeStruct((n, Wx.shape[1]), jnp.float32),
    )(x, Wx)


def _sc_gather_rows(xw, src):
    e = src.shape[0]
    d = xw.shape[1]
    c = 400
    per_w = e // _NW
    steps = per_w // c
    mesh = plsc.VectorSubcoreMesh(core_axis_name="c", subcore_axis_name="s")

    @functools.partial(
        pl.kernel,
        out_type=jax.ShapeDtypeStruct((e, d), jnp.float32),
        mesh=mesh,
        scratch_types=[
            pltpu.VMEM((c,), jnp.int32),
            pltpu.VMEM((c,), jnp.int32),
            pltpu.VMEM((c, d), jnp.float32),
            pltpu.VMEM((c, d), jnp.float32),
            pltpu.SemaphoreType.DMA,
            pltpu.SemaphoreType.DMA,
        ],
    )
    def k(xw_hbm, src_hbm, xwg_hbm, idx0, idx1, rows0, rows1, sem0, sem1):
        wid = lax.axis_index("s") * _NC + lax.axis_index("c")
        base = wid * per_w
        idxb = (idx0, idx1)
        rowsb = (rows0, rows1)
        semb = (sem0, sem1)

        pltpu.sync_copy(src_hbm.at[pl.ds(base, c)], idx0)
        handles = {0: pltpu.async_copy(xw_hbm.at[idx0], rows0, sem0)}
        for j in range(steps):
            b = j % 2
            handles[j].wait()
            if j + 1 < steps:
                nb = (j + 1) % 2
                pltpu.sync_copy(src_hbm.at[pl.ds(base + (j + 1) * c, c)],
                                idxb[nb])
                handles[j + 1] = pltpu.async_copy(xw_hbm.at[idxb[nb]],
                                                  rowsb[nb], semb[nb])
            pltpu.sync_copy(rowsb[b], xwg_hbm.at[pl.ds(base + j * c, c)])

    return k(xw, src)


def _sc_gather_dvec(pos_flat, src, dst):
    e = src.shape[0]
    npts8 = pos_flat.shape[0]
    c = 2000
    per_w = e // _NW
    steps = per_w // c
    mesh = plsc.VectorSubcoreMesh(core_axis_name="c", subcore_axis_name="s")

    @functools.partial(
        pl.kernel,
        out_type=jax.ShapeDtypeStruct((3, _NW, per_w), jnp.float32),
        mesh=mesh,
        scratch_types=[
            pltpu.VMEM((npts8,), jnp.float32),
            pltpu.VMEM((c,), jnp.int32),
            pltpu.VMEM((c,), jnp.int32),
            pltpu.VMEM((c,), jnp.float32),
            pltpu.VMEM((c,), jnp.float32),
            pltpu.VMEM((c,), jnp.float32),
        ],
        compiler_params=pltpu.CompilerParams(use_tc_tiling_on_sc=False,
                                             needs_layout_passes=False),
    )
    def k(pos_hbm, src_hbm, dst_hbm, dv_hbm, pos_v, idxs_v, idxd_v,
          dx_v, dy_v, dz_v):
        wid = lax.axis_index("s") * _NC + lax.axis_index("c")
        base = wid * per_w
        pltpu.sync_copy(pos_hbm, pos_v)

        @pl.loop(0, steps)
        def _(i):
            off = base + i * c
            pltpu.sync_copy(src_hbm.at[pl.ds(off, c)], idxs_v)
            pltpu.sync_copy(dst_hbm.at[pl.ds(off, c)], idxd_v)

            @pl.loop(0, c // 16)
            def _(g):
                sl = pl.ds(g * 16, 16)
                s8 = idxs_v[sl] * 8
                d8 = idxd_v[sl] * 8
                dx_v.at[sl][...] = (plsc.load_gather(pos_v, [d8])
                                    - plsc.load_gather(pos_v, [s8]))
                dy_v.at[sl][...] = (plsc.load_gather(pos_v, [d8 + 1])
                                    - plsc.load_gather(pos_v, [s8 + 1]))
                dz_v.at[sl][...] = (plsc.load_gather(pos_v, [d8 + 2])
                                    - plsc.load_gather(pos_v, [s8 + 2]))

            pltpu.sync_copy(dx_v, dv_hbm.at[0, wid, pl.ds(i * c, c)])
            pltpu.sync_copy(dy_v, dv_hbm.at[1, wid, pl.ds(i * c, c)])
            pltpu.sync_copy(dz_v, dv_hbm.at[2, wid, pl.ds(i * c, c)])

    return k(pos_flat, src, dst)


def _tc_message(xwg, dxa, dya, dza, W1T, b1T, W2, b2, Wy16):
    e, d = xwg.shape
    nb, _, b = dxa.shape
    s3 = np.float32(np.sqrt(3.0))
    dn = (((0,), (0,)), ((), ()))

    def body(xwg_ref, dx_ref, dy_ref, dz_ref, w1_ref, b1_ref, w2_ref,
             b2_ref, wy_ref, o_ref):
        dx = dx_ref[0]
        dy = dy_ref[0]
        dz = dz_ref[0]
        d2 = dx * dx + dy * dy + dz * dz
        ln = jnp.maximum(jnp.sqrt(d2), 1e-8)
        inv = 1.0 / ln
        ex = dx * inv
        ey = dy * inv
        ez = dz * inv
        Yl = jnp.concatenate(
            [
                jnp.ones_like(ex),
                ex, ey, ez,
                s3 * ex * ey,
                s3 * ey * ez,
                0.5 * (3.0 * ez * ez - 1.0),
                s3 * ex * ez,
                (s3 / 2.0) * (ex * ex - ey * ey),
                jnp.zeros((7, b), jnp.float32),
            ],
            axis=0,
        )
        yw = lax.dot_general(Yl, wy_ref[...], dn, precision=None)
        hl = jax.nn.silu(w1_ref[...] * ln + b1_ref[...])
        w = lax.dot_general(hl, w2_ref[...], dn, precision=None) + b2_ref[...]
        o_ref[...] = xwg_ref[...] * (yw * w)

    return pl.pallas_call(
        body,
        grid=(nb,),
        in_specs=[
            pl.BlockSpec((b, d), lambda i: (i, 0)),
            pl.BlockSpec((1, 1, b), lambda i: (i, 0, 0)),
            pl.BlockSpec((1, 1, b), lambda i: (i, 0, 0)),
            pl.BlockSpec((1, 1, b), lambda i: (i, 0, 0)),
            pl.BlockSpec((64, 1), lambda i: (0, 0)),
            pl.BlockSpec((64, 1), lambda i: (0, 0)),
            pl.BlockSpec((64, 128), lambda i: (0, 0)),
            pl.BlockSpec((1, 128), lambda i: (0, 0)),
            pl.BlockSpec((16, 128), lambda i: (0, 0)),
        ],
        out_specs=pl.BlockSpec((b, d), lambda i: (i, 0)),
        out_shape=jax.ShapeDtypeStruct((e, d), jnp.float32),
    )(xwg, dxa, dya, dza, W1T, b1T, W2, b2, Wy16)


def _sc_scatter(m_list, dst, n, e0):
    nm = len(m_list)
    ec, d = m_list[0].shape
    c = 80
    per_sub = ec // _NC // _NS
    steps = per_sub // c
    wb_rows = 1000
    zb = 40
    mesh = plsc.VectorSubcoreMesh(core_axis_name="c", subcore_axis_name="s")

    @functools.partial(
        pl.kernel,
        out_type=jax.ShapeDtypeStruct((_NC, n, d), jnp.float32),
        mesh=mesh,
        scratch_types=[
            pltpu.VMEM((c,), jnp.int32),
            pltpu.VMEM((c,), jnp.int32),
            pltpu.VMEM((c, d), jnp.float32),
            pltpu.VMEM((c, d), jnp.float32),
            pltpu.VMEM((zb, d), jnp.float32),
            pltpu.VMEM_SHARED((n, d), jnp.float32),
            pltpu.SemaphoreType.DMA,
            pltpu.SemaphoreType.DMA,
        ],
    )
    def k(*refs):
        m_hbms = refs[:nm]
        (dst_hbm, out_hbm, idx0, idx1, rows0, rows1, zero_v, acc_sh,
         sem0, sem1) = refs[nm:]
        cid = lax.axis_index("c")
        sid = lax.axis_index("s")
        zvec = jnp.zeros((16,), jnp.float32)
        idxb = (idx0, idx1)
        rowsb = (rows0, rows1)
        semb = (sem0, sem1)

        @pl.loop(0, zb)
        def _(r):
            @pl.loop(0, d // 16)
            def _(j):
                zero_v.at[r, pl.ds(j * 16, 16)][...] = zvec

        @pl.when(sid < n // wb_rows)
        def _():
            @pl.loop(0, wb_rows // zb)
            def _(bk):
                pltpu.sync_copy(zero_v,
                                acc_sh.at[pl.ds(sid * wb_rows + bk * zb, zb)])

        plsc.subcore_barrier()

        local0 = cid * (ec // _NC) + sid * per_sub
        iters = [(m_hbms[mi], mi, i) for mi in range(nm)
                 for i in range(steps)]

        def start_load(j, b):
            m_hbm, mi, i = iters[j]
            loc = local0 + i * c
            gof = e0 + mi * ec + loc
            return (pltpu.async_copy(dst_hbm.at[pl.ds(gof, c)], idxb[b],
                                     semb[b]),
                    pltpu.async_copy(m_hbm.at[pl.ds(loc, c)], rowsb[b],
                                     semb[b]))

        handles = {0: start_load(0, 0)}
        for j in range(len(iters)):
            b = j % 2
            h1, h2 = handles[j]
            h1.wait()
            h2.wait()
            if j + 1 < len(iters):
                handles[j + 1] = start_load(j + 1, (j + 1) % 2)
            pltpu.sync_copy(rowsb[b], acc_sh.at[idxb[b]], add=True)

        plsc.subcore_barrier()

        @pl.when(sid < n // wb_rows)
        def _():
            pltpu.sync_copy(acc_sh.at[pl.ds(sid * wb_rows, wb_rows)],
                            out_hbm.at[cid, pl.ds(sid * wb_rows, wb_rows)])

    return k(*m_list, dst)


def _tc_head(parts_list, Ws, Wns, Wg):
    np_ = len(parts_list)
    _, n, d = parts_list[0].shape
    b = 1000

    def body(*refs):
        p_refs = refs[:np_]
        ws_ref, wns_ref, wg_ref, o_ref = refs[np_:]
        out = p_refs[0][0] + p_refs[0][1]
        for p in p_refs[1:]:
            out = out + p[0] + p[1]
        s = jax.nn.silu(jnp.dot(out, ws_ref[...]))
        ns = jnp.dot(out, wns_ref[...])
        g = jax.nn.sigmoid(jnp.dot(out, wg_ref[...]))
        i0 = lax.broadcasted_iota(jnp.int32, (32, 96), 0)
        i1 = lax.broadcasted_iota(jnp.int32, (32, 96), 1)
        rep = (i0 == i1 // 3).astype(jnp.float32)
        gr = jnp.dot(g, rep, precision=_H)
        o_ref[...] = jnp.concatenate([s, gr * ns], axis=1)

    return pl.pallas_call(
        body,
        grid=(n // b,),
        in_specs=(
            [pl.BlockSpec((2, b, d), lambda i: (0, i, 0))] * np_
            + [
                pl.BlockSpec((128, 32), lambda i: (0, 0)),
                pl.BlockSpec((128, 96), lambda i: (0, 0)),
                pl.BlockSpec((128, 32), lambda i: (0, 0)),
            ]
        ),
        out_specs=pl.BlockSpec((b, d), lambda i: (i, 0)),
        out_shape=jax.ShapeDtypeStruct((n, d), jnp.float32),
    )(*parts_list, Ws, Wns, Wg)


def kernel(x, edge_index, pos, W1, b1, W2, b2, Wx, Wy, Ws, Wns, Wg):
    n = x.shape[0]
    e = edge_index.shape[1]
    be = 2560
    nk = 5
    ec = e // nk
    src = edge_index[0]
    dst = edge_index[1]
    pos_flat = jnp.pad(pos, ((0, 0), (0, 5))).reshape(-1)
    Wy16 = jnp.pad(Wy, ((0, 7), (0, 0)))
    W1T = W1.reshape(-1, 1)
    b1T = b1.reshape(-1, 1)
    b2r = b2.reshape(1, -1)

    xw = _tc_matmul(x, Wx)
    dv = _sc_gather_dvec(pos_flat, src, dst).reshape(3, e)
    ms = []
    for k in range(nk):
        srck = lax.slice(src, (k * ec,), ((k + 1) * ec,))
        xwg = _sc_gather_rows(xw, srck)
        dxa = lax.slice(dv[0], (k * ec,), ((k + 1) * ec,)).reshape(
            ec // be, 1, be)
        dya = lax.slice(dv[1], (k * ec,), ((k + 1) * ec,)).reshape(
            ec // be, 1, be)
        dza = lax.slice(dv[2], (k * ec,), ((k + 1) * ec,)).reshape(
            ec // be, 1, be)
        ms.append(_tc_message(xwg, dxa, dya, dza, W1T, b1T, W2, b2r, Wy16))
    parts_list = [
        _sc_scatter(ms[:4], dst, n, 0),
        _sc_scatter(ms[4:], dst, n, 4 * ec),
    ]
    return _tc_head(parts_list, Ws, Wns, Wg)


# speedup vs baseline: 1.5250x; 1.0316x over previous
import functools

import jax
import jax.numpy as jnp
import numpy as np
from jax import lax
from jax.experimental import pallas as pl
from jax.experimental.pallas import tpu as pltpu
from jax.experimental.pallas import tpu_sc as plsc

_NC = 2
_NS = 16
_NW = _NC * _NS
_H = jax.lax.Precision.HIGHEST


def _tc_matmul(x, Wx):
    n, d = x.shape
    b = 1000

    def body(x_ref, w_ref, o_ref):
        o_ref[...] = jnp.dot(x_ref[...], w_ref[...], precision=_H)

    return pl.pallas_call(
        body,
        grid=(n // b,),
        in_specs=[
            pl.BlockSpec((b, d), lambda i: (i, 0)),
            pl.BlockSpec(Wx.shape, lambda i: (0, 0)),
        ],
        out_specs=pl.BlockSpec((b, Wx.shape[1]), lambda i: (i, 0)),
        out_shape=jax.ShapeDtypeStruct((n, Wx.shape[1]), jnp.float32),
    )(x, Wx)


def _sc_gather_rows(xw, src):
    e = src.shape[0]
    d = xw.shape[1]
    c = 400
    per_w = e // _NW
    steps = per_w // c
    mesh = plsc.VectorSubcoreMesh(core_axis_name="c", subcore_axis_name="s")

    @functools.partial(
        pl.kernel,
        out_type=jax.ShapeDtypeStruct((e, d), jnp.float32),
        mesh=mesh,
        scratch_types=[
            pltpu.VMEM((c,), jnp.int32),
            pltpu.VMEM((c,), jnp.int32),
            pltpu.VMEM((c, d), jnp.float32),
            pltpu.VMEM((c, d), jnp.float32),
            pltpu.SemaphoreType.DMA,
            pltpu.SemaphoreType.DMA,
        ],
    )
    def k(xw_hbm, src_hbm, xwg_hbm, idx0, idx1, rows0, rows1, sem0, sem1):
        wid = lax.axis_index("s") * _NC + lax.axis_index("c")
        base = wid * per_w
        idxb = (idx0, idx1)
        rowsb = (rows0, rows1)
        semb = (sem0, sem1)

        pltpu.sync_copy(src_hbm.at[pl.ds(base, c)], idx0)
        handles = {0: pltpu.async_copy(xw_hbm.at[idx0], rows0, sem0)}
        for j in range(steps):
            b = j % 2
            handles[j].wait()
            if j + 1 < steps:
                nb = (j + 1) % 2
                pltpu.sync_copy(src_hbm.at[pl.ds(base + (j + 1) * c, c)],
                                idxb[nb])
                handles[j + 1] = pltpu.async_copy(xw_hbm.at[idxb[nb]],
                                                  rowsb[nb], semb[nb])
            pltpu.sync_copy(rowsb[b], xwg_hbm.at[pl.ds(base + j * c, c)])

    return k(xw, src)


def _sc_gather_dvec(pos_flat, src, dst):
    e = src.shape[0]
    npts8 = pos_flat.shape[0]
    c = 2000
    per_w = e // _NW
    steps = per_w // c
    mesh = plsc.VectorSubcoreMesh(core_axis_name="c", subcore_axis_name="s")

    @functools.partial(
        pl.kernel,
        out_type=jax.ShapeDtypeStruct((3, _NW, per_w), jnp.float32),
        mesh=mesh,
        scratch_types=[
            pltpu.VMEM((npts8,), jnp.float32),
            pltpu.VMEM((c,), jnp.int32),
            pltpu.VMEM((c,), jnp.int32),
            pltpu.VMEM((c,), jnp.float32),
            pltpu.VMEM((c,), jnp.float32),
            pltpu.VMEM((c,), jnp.float32),
        ],
        compiler_params=pltpu.CompilerParams(use_tc_tiling_on_sc=False,
                                             needs_layout_passes=False),
    )
    def k(pos_hbm, src_hbm, dst_hbm, dv_hbm, pos_v, idxs_v, idxd_v,
          dx_v, dy_v, dz_v):
        wid = lax.axis_index("s") * _NC + lax.axis_index("c")
        base = wid * per_w
        pltpu.sync_copy(pos_hbm, pos_v)

        @pl.loop(0, steps)
        def _(i):
            off = base + i * c
            pltpu.sync_copy(src_hbm.at[pl.ds(off, c)], idxs_v)
            pltpu.sync_copy(dst_hbm.at[pl.ds(off, c)], idxd_v)

            @pl.loop(0, c // 16)
            def _(g):
                sl = pl.ds(g * 16, 16)
                s8 = idxs_v[sl] * 8
                d8 = idxd_v[sl] * 8
                dx_v.at[sl][...] = (plsc.load_gather(pos_v, [d8])
                                    - plsc.load_gather(pos_v, [s8]))
                dy_v.at[sl][...] = (plsc.load_gather(pos_v, [d8 + 1])
                                    - plsc.load_gather(pos_v, [s8 + 1]))
                dz_v.at[sl][...] = (plsc.load_gather(pos_v, [d8 + 2])
                                    - plsc.load_gather(pos_v, [s8 + 2]))

            pltpu.sync_copy(dx_v, dv_hbm.at[0, wid, pl.ds(i * c, c)])
            pltpu.sync_copy(dy_v, dv_hbm.at[1, wid, pl.ds(i * c, c)])
            pltpu.sync_copy(dz_v, dv_hbm.at[2, wid, pl.ds(i * c, c)])

    return k(pos_flat, src, dst)


def _tc_message(xwg, dxa, dya, dza, W1T, b1T, W2, b2, Wy16):
    e, d = xwg.shape
    nb, _, b = dxa.shape
    s3 = np.float32(np.sqrt(3.0))
    dn = (((0,), (0,)), ((), ()))

    def body(xwg_ref, dx_ref, dy_ref, dz_ref, w1_ref, b1_ref, w2_ref,
             b2_ref, wy_ref, o_ref):
        dx = dx_ref[0]
        dy = dy_ref[0]
        dz = dz_ref[0]
        d2 = dx * dx + dy * dy + dz * dz
        ln = jnp.maximum(jnp.sqrt(d2), 1e-8)
        inv = 1.0 / ln
        ex = dx * inv
        ey = dy * inv
        ez = dz * inv
        Yl = jnp.concatenate(
            [
                jnp.ones_like(ex),
                ex, ey, ez,
                s3 * ex * ey,
                s3 * ey * ez,
                0.5 * (3.0 * ez * ez - 1.0),
                s3 * ex * ez,
                (s3 / 2.0) * (ex * ex - ey * ey),
                jnp.zeros((7, b), jnp.float32),
            ],
            axis=0,
        )
        yw = lax.dot_general(Yl, wy_ref[...], dn, precision=None)
        hl = jax.nn.silu(w1_ref[...] * ln + b1_ref[...])
        w = lax.dot_general(hl, w2_ref[...], dn, precision=None) + b2_ref[...]
        o_ref[...] = xwg_ref[...] * (yw * w)

    return pl.pallas_call(
        body,
        grid=(nb,),
        in_specs=[
            pl.BlockSpec((b, d), lambda i: (i, 0)),
            pl.BlockSpec((1, 1, b), lambda i: (i, 0, 0)),
            pl.BlockSpec((1, 1, b), lambda i: (i, 0, 0)),
            pl.BlockSpec((1, 1, b), lambda i: (i, 0, 0)),
            pl.BlockSpec((64, 1), lambda i: (0, 0)),
            pl.BlockSpec((64, 1), lambda i: (0, 0)),
            pl.BlockSpec((64, 128), lambda i: (0, 0)),
            pl.BlockSpec((1, 128), lambda i: (0, 0)),
            pl.BlockSpec((16, 128), lambda i: (0, 0)),
        ],
        out_specs=pl.BlockSpec((b, d), lambda i: (i, 0)),
        out_shape=jax.ShapeDtypeStruct((e, d), jnp.float32),
    )(xwg, dxa, dya, dza, W1T, b1T, W2, b2, Wy16)


def _sc_scatter(m_list, dst, n, e0):
    nm = len(m_list)
    ec, d = m_list[0].shape
    c = 80
    per_sub = ec // _NC // _NS
    steps = per_sub // c
    wb_rows = 1000
    zb = 40
    mesh = plsc.VectorSubcoreMesh(core_axis_name="c", subcore_axis_name="s")

    @functools.partial(
        pl.kernel,
        out_type=jax.ShapeDtypeStruct((_NC, n, d), jnp.float32),
        mesh=mesh,
        scratch_types=[
            pltpu.VMEM((c,), jnp.int32),
            pltpu.VMEM((c,), jnp.int32),
            pltpu.VMEM((c, d), jnp.float32),
            pltpu.VMEM((c, d), jnp.float32),
            pltpu.VMEM((zb, d), jnp.float32),
            pltpu.VMEM_SHARED((n, d), jnp.float32),
            pltpu.SemaphoreType.DMA,
            pltpu.SemaphoreType.DMA,
        ],
    )
    def k(*refs):
        m_hbms = refs[:nm]
        (dst_hbm, out_hbm, idx0, idx1, rows0, rows1, zero_v, acc_sh,
         sem0, sem1) = refs[nm:]
        cid = lax.axis_index("c")
        sid = lax.axis_index("s")
        zvec = jnp.zeros((16,), jnp.float32)
        idxb = (idx0, idx1)
        rowsb = (rows0, rows1)
        semb = (sem0, sem1)

        @pl.loop(0, zb)
        def _(r):
            @pl.loop(0, d // 16)
            def _(j):
                zero_v.at[r, pl.ds(j * 16, 16)][...] = zvec

        @pl.when(sid < n // wb_rows)
        def _():
            @pl.loop(0, wb_rows // zb)
            def _(bk):
                pltpu.sync_copy(zero_v,
                                acc_sh.at[pl.ds(sid * wb_rows + bk * zb, zb)])

        plsc.subcore_barrier()

        local0 = cid * (ec // _NC) + sid * per_sub
        iters = [(m_hbms[mi], mi, i) for mi in range(nm)
                 for i in range(steps)]

        def start_load(j, b):
            m_hbm, mi, i = iters[j]
            loc = local0 + i * c
            gof = e0 + mi * ec + loc
            return (pltpu.async_copy(dst_hbm.at[pl.ds(gof, c)], idxb[b],
                                     semb[b]),
                    pltpu.async_copy(m_hbm.at[pl.ds(loc, c)], rowsb[b],
                                     semb[b]))

        handles = {0: start_load(0, 0)}
        for j in range(len(iters)):
            b = j % 2
            h1, h2 = handles[j]
            h1.wait()
            h2.wait()
            if j + 1 < len(iters):
                handles[j + 1] = start_load(j + 1, (j + 1) % 2)
            pltpu.sync_copy(rowsb[b], acc_sh.at[idxb[b]], add=True)

        plsc.subcore_barrier()

        @pl.when(sid < n // wb_rows)
        def _():
            pltpu.sync_copy(acc_sh.at[pl.ds(sid * wb_rows, wb_rows)],
                            out_hbm.at[cid, pl.ds(sid * wb_rows, wb_rows)])

    return k(*m_list, dst)


def _tc_head(parts_list, Ws, Wns, Wg):
    np_ = len(parts_list)
    _, n, d = parts_list[0].shape
    b = 1000

    def body(*refs):
        p_refs = refs[:np_]
        ws_ref, wns_ref, wg_ref, o_ref = refs[np_:]
        out = p_refs[0][0] + p_refs[0][1]
        for p in p_refs[1:]:
            out = out + p[0] + p[1]
        s = jax.nn.silu(jnp.dot(out, ws_ref[...]))
        ns = jnp.dot(out, wns_ref[...])
        g = jax.nn.sigmoid(jnp.dot(out, wg_ref[...]))
        i0 = lax.broadcasted_iota(jnp.int32, (32, 96), 0)
        i1 = lax.broadcasted_iota(jnp.int32, (32, 96), 1)
        rep = (i0 == i1 // 3).astype(jnp.float32)
        gr = jnp.dot(g, rep, precision=_H)
        o_ref[...] = jnp.concatenate([s, gr * ns], axis=1)

    return pl.pallas_call(
        body,
        grid=(n // b,),
        in_specs=(
            [pl.BlockSpec((2, b, d), lambda i: (0, i, 0))] * np_
            + [
                pl.BlockSpec((128, 32), lambda i: (0, 0)),
                pl.BlockSpec((128, 96), lambda i: (0, 0)),
                pl.BlockSpec((128, 32), lambda i: (0, 0)),
            ]
        ),
        out_specs=pl.BlockSpec((b, d), lambda i: (i, 0)),
        out_shape=jax.ShapeDtypeStruct((n, d), jnp.float32),
    )(*parts_list, Ws, Wns, Wg)


def kernel(x, edge_index, pos, W1, b1, W2, b2, Wx, Wy, Ws, Wns, Wg):
    n = x.shape[0]
    e = edge_index.shape[1]
    be = 2560
    nk = 5
    ec = e // nk
    src = edge_index[0]
    dst = edge_index[1]
    pos_flat = jnp.pad(pos, ((0, 0), (0, 5))).reshape(-1)
    Wy16 = jnp.pad(Wy, ((0, 7), (0, 0)))
    W1T = W1.reshape(-1, 1)
    b1T = b1.reshape(-1, 1)
    b2r = b2.reshape(1, -1)

    xw = _tc_matmul(x, Wx)
    e_a = 2 * ec
    dv_a = _sc_gather_dvec(pos_flat,
                           lax.slice(src, (0,), (e_a,)),
                           lax.slice(dst, (0,), (e_a,))).reshape(3, e_a)
    dv_b = _sc_gather_dvec(pos_flat,
                           lax.slice(src, (e_a,), (e,)),
                           lax.slice(dst, (e_a,), (e,))).reshape(3, e - e_a)
    ms = []
    for k in range(nk):
        srck = lax.slice(src, (k * ec,), ((k + 1) * ec,))
        xwg = _sc_gather_rows(xw, srck)
        dvk, off = (dv_a, k * ec) if k < 2 else (dv_b, (k - 2) * ec)
        dxa = lax.slice(dvk[0], (off,), (off + ec,)).reshape(ec // be, 1, be)
        dya = lax.slice(dvk[1], (off,), (off + ec,)).reshape(ec // be, 1, be)
        dza = lax.slice(dvk[2], (off,), (off + ec,)).reshape(ec // be, 1, be)
        ms.append(_tc_message(xwg, dxa, dya, dza, W1T, b1T, W2, b2r, Wy16))
    parts_list = [
        _sc_scatter(ms[:2], dst, n, 0),
        _sc_scatter(ms[2:4], dst, n, 2 * ec),
        _sc_scatter(ms[4:], dst, n, 4 * ec),
    ]
    return _tc_head(parts_list, Ws, Wns, Wg)
